# baseline (speedup 1.0000x reference)
import functools

import jax
import jax.numpy as jnp
from jax import lax
from jax.experimental import pallas as pl
from jax.experimental.pallas import tpu as pltpu
from jax.experimental.pallas import tpu_sc as plsc

EOS_ID = 49407
SEQ = 77
DIM = 512
NSEQ = 4096
LANES = 16
NC = 2
NS = 16
NW = NC * NS
SPW = NSEQ // NW
NB = 4
QTR = SPW // NB


def _emb_body(ids_t, table, pos, out, idx, pos_v, buf,
              gs0, gs1, gs2, gs3, ws0, ws1, ws2, ws3):
    gsem = (gs0, gs1, gs2, gs3)
    wsem = (ws0, ws1, ws2, ws3)
    wid = lax.axis_index("s") * NC + lax.axis_index("c")
    base = wid * SPW
    pltpu.sync_copy(pos, pos_v)
    pltpu.sync_copy(ids_t.at[:, pl.ds(base, SPW)], idx)

    def start_gather(h, p):
        pltpu.async_copy(table.at[idx.at[p, pl.ds(h * QTR, QTR)]],
                         buf.at[h], gsem[h])

    def wait_gather(h):
        pltpu.make_async_copy(table.at[pl.ds(0, QTR)], buf.at[h],
                              gsem[h]).wait()

    def start_write(h, p):
        pltpu.async_copy(buf.at[h], out.at[p, pl.ds(base + h * QTR, QTR)],
                         wsem[h])

    def wait_write(h):
        pltpu.make_async_copy(buf.at[0], out.at[0, pl.ds(0, QTR)],
                              wsem[h]).wait()

    def addpass(h, p):
        for cb in range(DIM // LANES // 8):
            pvs = [pos_v[p, pl.ds((cb * 8 + j) * LANES, LANES)]
                   for j in range(8)]

            def row_body(r, carry):
                for j in range(8):
                    sl = pl.ds((cb * 8 + j) * LANES, LANES)
                    buf[h, r, sl] = buf[h, r, sl] + pvs[j]
                return carry

            lax.fori_loop(0, QTR, row_body, 0)

    start_gather(0, 0)
    start_gather(1, 0)
    start_gather(2, 0)

    def ploop(p, carry):
        for h in range(NB):
            wait_gather(h)
            addpass(h, p)
            start_write(h, p)
            hp = (h + 3) % NB
            if h == 0:
                @pl.when(p > 0)
                def _():
                    wait_write(hp)

                start_gather(hp, p)
            else:
                wait_write(hp)

                @pl.when(p < SEQ - 1)
                def _():
                    start_gather(hp, p + 1)

        return carry

    lax.fori_loop(0, SEQ, ploop, 0)
    wait_write(NB - 1)


def _mask_body(ids_ref, out_ref):
    out_ref[...] = ids_ref[...] == EOS_ID


def kernel(input_ids, embedding_table, positional_embedding):
    ids_t = jnp.transpose(input_ids)
    mesh = plsc.VectorSubcoreMesh(core_axis_name="c", subcore_axis_name="s")
    emb = functools.partial(
        pl.kernel,
        mesh=mesh,
        out_type=jax.ShapeDtypeStruct((SEQ, NSEQ, DIM), jnp.float32),
        scratch_types=[
            pltpu.VMEM((SEQ, SPW), jnp.int32),
            pltpu.VMEM((SEQ, DIM), jnp.float32),
            pltpu.VMEM((NB, QTR, DIM), jnp.float32),
            pltpu.SemaphoreType.DMA,
            pltpu.SemaphoreType.DMA,
            pltpu.SemaphoreType.DMA,
            pltpu.SemaphoreType.DMA,
            pltpu.SemaphoreType.DMA,
            pltpu.SemaphoreType.DMA,
            pltpu.SemaphoreType.DMA,
            pltpu.SemaphoreType.DMA,
        ],
    )(_emb_body)
    tokens_t = emb(ids_t, embedding_table, positional_embedding)
    tokens = jnp.transpose(tokens_t, (1, 0, 2))
    mask = pl.pallas_call(
        _mask_body,
        out_shape=jax.ShapeDtypeStruct((NSEQ, SEQ), jnp.bool_),
    )(input_ids)
    return (tokens, mask)
